# baseline (speedup 1.0000x reference)
import functools

import jax
import jax.numpy as jnp
from jax.experimental import pallas as pl
from jax.experimental.pallas import tpu as pltpu

_BINS = 4096
_ADIM = 8
_OUT_DIM = _BINS * (_ADIM + 1)
_BK = 128
_BS = 16

def _gumbel_noise():
    return jax.random.gumbel(
        jax.random.key(42), (_BS, _BINS), jnp.float32
    )


_NJ = 8
_SLABW = _OUT_DIM // _NJ


def _fused_body(
    x_ref,
    w_ref,
    b_ref,
    gmb_ref,
    obins_ref,
    ores_ref,
    osel_ref,
    oselres_ref,
    *,
    nsteps,
):
    k = pl.program_id(0)
    j = pl.program_id(1)
    xk = x_ref[:, pl.ds(k * _BK, _BK)]
    xk16 = xk.astype(jnp.bfloat16)
    wk = w_ref[...]

    def acc_bins(val, lo, width):
        @pl.when(k == 0)
        def _():
            obins_ref[:, lo : lo + width] = (
                val + b_ref[:, lo : lo + width]
            )

        @pl.when(k != 0)
        def _():
            obins_ref[:, lo : lo + width] = (
                obins_ref[:, lo : lo + width] + val
            )

    def acc_res(val, lo, width):
        @pl.when(k == 0)
        def _():
            ores_ref[:, lo : lo + width] = (
                val + b_ref[:, _BINS + lo : _BINS + lo + width]
            )

        @pl.when(k != 0)
        def _():
            ores_ref[:, lo : lo + width] = (
                ores_ref[:, lo : lo + width] + val
            )

    for jj in range(_NJ):
        lo = jj * _SLABW

        @pl.when(j == jj)
        def _(jj=jj, lo=lo):
            if lo < _BINS:
                nb = min(_BINS - lo, _SLABW)
                acc_bins(
                    jnp.dot(
                        xk,
                        wk[:, :nb],
                        preferred_element_type=jnp.float32,
                    ),
                    lo,
                    nb,
                )
                if nb < _SLABW:
                    acc_res(
                        jnp.dot(
                            xk16,
                            wk[:, nb:].astype(jnp.bfloat16),
                            preferred_element_type=jnp.float32,
                        ),
                        lo + nb - _BINS,
                        _SLABW - nb,
                    )
            else:
                acc_res(
                    jnp.dot(
                        xk16,
                        wk.astype(jnp.bfloat16),
                        preferred_element_type=jnp.float32,
                    ),
                    lo - _BINS,
                    _SLABW,
                )

    @pl.when((k == nsteps - 1) & (j == _NJ - 1))
    def _():
        z = obins_ref[...] + gmb_ref[...]
        sel = jnp.argmax(z, axis=-1).astype(jnp.int32)
        osel_ref[...] = sel[:, None]
        cols = jax.lax.broadcasted_iota(jnp.int32, (_BS, _BINS * _ADIM), 1)
        resid = ores_ref[...]
        parts = []
        for c in range(_ADIM):
            m = cols == sel[:, None] * _ADIM + c
            parts.append(
                jnp.sum(jnp.where(m, resid, 0.0), axis=1, keepdims=True)
            )
        oselres_ref[...] = jnp.concatenate(parts, axis=1)


def kernel(transformer_logits, W, b):
    batch, seq, num_bins = transformer_logits.shape
    bs = batch * seq
    x2d = transformer_logits.reshape(bs, num_bins)
    b2d = b.reshape(1, _OUT_DIM)
    gumbel = _gumbel_noise()

    nsteps = num_bins // _BK
    bins_logits, resid, sel, selres = pl.pallas_call(
        functools.partial(_fused_body, nsteps=nsteps),
        grid=(nsteps, _NJ),
        in_specs=[
            pl.BlockSpec((bs, num_bins), lambda k, j: (0, 0)),
            pl.BlockSpec((_BK, _SLABW), lambda k, j: (k, j)),
            pl.BlockSpec((1, _OUT_DIM), lambda k, j: (0, 0)),
            pl.BlockSpec((bs, _BINS), lambda k, j: (0, 0)),
        ],
        out_specs=(
            pl.BlockSpec((bs, _BINS), lambda k, j: (0, 0)),
            pl.BlockSpec((bs, _OUT_DIM - _BINS), lambda k, j: (0, 0)),
            pl.BlockSpec((bs, 1), lambda k, j: (0, 0)),
            pl.BlockSpec((bs, _ADIM), lambda k, j: (0, 0)),
        ),
        out_shape=(
            jax.ShapeDtypeStruct((bs, _BINS), jnp.float32),
            jax.ShapeDtypeStruct((bs, _OUT_DIM - _BINS), jnp.float32),
            jax.ShapeDtypeStruct((bs, 1), jnp.int32),
            jax.ShapeDtypeStruct((bs, _ADIM), jnp.float32),
        ),
        compiler_params=pltpu.CompilerParams(
            dimension_semantics=("arbitrary", "arbitrary")
        ),
    )(x2d, W, b2d, gumbel)

    return (
        sel.reshape(batch, seq, 1),
        selres.reshape(batch, seq, _ADIM),
        resid.reshape(batch, seq, num_bins, _ADIM),
        bins_logits.reshape(batch, seq, num_bins),
    )


# speedup vs baseline: 1.3040x; 1.3040x over previous
import functools

import jax
import jax.numpy as jnp
from jax.experimental import pallas as pl
from jax.experimental.pallas import tpu as pltpu

_BINS = 4096
_ADIM = 8
_OUT_DIM = _BINS * (_ADIM + 1)
_BK = 128
_BS = 16

def _gumbel_noise():
    return jax.random.gumbel(
        jax.random.key(42), (_BS, _BINS), jnp.float32
    )


_NJ = 4
_SLABW = _OUT_DIM // _NJ


def _fused_body(
    x_ref,
    w_ref,
    b_ref,
    gmb_ref,
    obins_ref,
    ores_ref,
    osel_ref,
    oselres_ref,
    *,
    nsteps,
):
    k = pl.program_id(0)
    j = pl.program_id(1)
    xk = x_ref[:, pl.ds(k * _BK, _BK)]
    xk16 = xk.astype(jnp.bfloat16)
    wk = w_ref[...]

    def acc_bins(val, lo, width):
        @pl.when(k == 0)
        def _():
            obins_ref[:, lo : lo + width] = (
                val + b_ref[:, lo : lo + width]
            )

        @pl.when(k != 0)
        def _():
            obins_ref[:, lo : lo + width] = (
                obins_ref[:, lo : lo + width] + val
            )

    def acc_res(val, lo, width):
        @pl.when(k == 0)
        def _():
            ores_ref[:, lo : lo + width] = (
                val + b_ref[:, _BINS + lo : _BINS + lo + width]
            )

        @pl.when(k != 0)
        def _():
            ores_ref[:, lo : lo + width] = (
                ores_ref[:, lo : lo + width] + val
            )

    for jj in range(_NJ):
        lo = jj * _SLABW

        @pl.when(j == jj)
        def _(jj=jj, lo=lo):
            if lo < _BINS:
                nb = min(_BINS - lo, _SLABW)
                acc_bins(
                    jnp.dot(
                        xk,
                        wk[:, :nb],
                        preferred_element_type=jnp.float32,
                    ),
                    lo,
                    nb,
                )
                if nb < _SLABW:
                    acc_res(
                        jnp.dot(
                            xk16,
                            wk[:, nb:].astype(jnp.bfloat16),
                            preferred_element_type=jnp.float32,
                        ),
                        lo + nb - _BINS,
                        _SLABW - nb,
                    )
            else:
                acc_res(
                    jnp.dot(
                        xk16,
                        wk.astype(jnp.bfloat16),
                        preferred_element_type=jnp.float32,
                    ),
                    lo - _BINS,
                    _SLABW,
                )

    @pl.when((k == nsteps - 1) & (j == _NJ - 1))
    def _():
        z = obins_ref[...] + gmb_ref[...]
        sel = jnp.argmax(z, axis=-1).astype(jnp.int32)
        osel_ref[...] = sel[:, None]
        cols = jax.lax.broadcasted_iota(jnp.int32, (_BS, _BINS * _ADIM), 1)
        resid = ores_ref[...]
        parts = []
        for c in range(_ADIM):
            m = cols == sel[:, None] * _ADIM + c
            parts.append(
                jnp.sum(jnp.where(m, resid, 0.0), axis=1, keepdims=True)
            )
        oselres_ref[...] = jnp.concatenate(parts, axis=1)


def kernel(transformer_logits, W, b):
    batch, seq, num_bins = transformer_logits.shape
    bs = batch * seq
    x2d = transformer_logits.reshape(bs, num_bins)
    b2d = b.reshape(1, _OUT_DIM)
    gumbel = _gumbel_noise()

    nsteps = num_bins // _BK
    bins_logits, resid, sel, selres = pl.pallas_call(
        functools.partial(_fused_body, nsteps=nsteps),
        grid=(nsteps, _NJ),
        in_specs=[
            pl.BlockSpec((bs, num_bins), lambda k, j: (0, 0)),
            pl.BlockSpec((_BK, _SLABW), lambda k, j: (k, j)),
            pl.BlockSpec((1, _OUT_DIM), lambda k, j: (0, 0)),
            pl.BlockSpec((bs, _BINS), lambda k, j: (0, 0)),
        ],
        out_specs=(
            pl.BlockSpec((bs, _BINS), lambda k, j: (0, 0)),
            pl.BlockSpec((bs, _OUT_DIM - _BINS), lambda k, j: (0, 0)),
            pl.BlockSpec((bs, 1), lambda k, j: (0, 0)),
            pl.BlockSpec((bs, _ADIM), lambda k, j: (0, 0)),
        ),
        out_shape=(
            jax.ShapeDtypeStruct((bs, _BINS), jnp.float32),
            jax.ShapeDtypeStruct((bs, _OUT_DIM - _BINS), jnp.float32),
            jax.ShapeDtypeStruct((bs, 1), jnp.int32),
            jax.ShapeDtypeStruct((bs, _ADIM), jnp.float32),
        ),
        compiler_params=pltpu.CompilerParams(
            dimension_semantics=("arbitrary", "arbitrary")
        ),
    )(x2d, W, b2d, gumbel)

    return (
        sel.reshape(batch, seq, 1),
        selres.reshape(batch, seq, _ADIM),
        resid.reshape(batch, seq, num_bins, _ADIM),
        bins_logits.reshape(batch, seq, num_bins),
    )


# speedup vs baseline: 1.5572x; 1.1942x over previous
import functools

import jax
import jax.numpy as jnp
from jax.experimental import pallas as pl
from jax.experimental.pallas import tpu as pltpu

_BINS = 4096
_ADIM = 8
_OUT_DIM = _BINS * (_ADIM + 1)
_BK = 128
_BS = 16

def _gumbel_noise():
    return jax.random.gumbel(
        jax.random.key(42), (_BS, _BINS), jnp.float32
    )


_HALFW = _OUT_DIM // 2
_RES0 = _HALFW - _BINS


def _fused_body(
    x_ref,
    w_ref,
    b_ref,
    gmb_ref,
    obins_ref,
    ores_ref,
    osel_ref,
    oselres_ref,
    *,
    nsteps,
):
    k = pl.program_id(0)
    j = pl.program_id(1)
    xk = x_ref[:, pl.ds(k * _BK, _BK)]
    xk16 = xk.astype(jnp.bfloat16)
    wk = w_ref[...]

    @pl.when(j == 0)
    def _():
        bins_part = jnp.dot(
            xk, wk[:, :_BINS], preferred_element_type=jnp.float32
        )
        res_part = jnp.dot(
            xk16,
            wk[:, _BINS:].astype(jnp.bfloat16),
            preferred_element_type=jnp.float32,
        )

        @pl.when(k == 0)
        def _():
            obins_ref[...] = bins_part + b_ref[:, :_BINS]
            ores_ref[:, :_RES0] = res_part + b_ref[:, _BINS:_HALFW]

        @pl.when(k != 0)
        def _():
            obins_ref[...] = obins_ref[...] + bins_part
            ores_ref[:, :_RES0] = ores_ref[:, :_RES0] + res_part

    @pl.when(j == 1)
    def _():
        res_part = jnp.dot(
            xk16,
            wk.astype(jnp.bfloat16),
            preferred_element_type=jnp.float32,
        )

        @pl.when(k == 0)
        def _():
            ores_ref[:, _RES0:] = res_part + b_ref[:, _HALFW:]

        @pl.when(k != 0)
        def _():
            ores_ref[:, _RES0:] = ores_ref[:, _RES0:] + res_part

    @pl.when((k == nsteps - 1) & (j == 1))
    def _():
        z = obins_ref[...] + gmb_ref[...]
        sel = jnp.argmax(z, axis=-1).astype(jnp.int32)
        osel_ref[...] = sel[:, None]
        cols = jax.lax.broadcasted_iota(jnp.int32, (_BS, _BINS * _ADIM), 1)
        resid = ores_ref[...]
        parts = []
        for c in range(_ADIM):
            m = cols == sel[:, None] * _ADIM + c
            parts.append(
                jnp.sum(jnp.where(m, resid, 0.0), axis=1, keepdims=True)
            )
        oselres_ref[...] = jnp.concatenate(parts, axis=1)


def kernel(transformer_logits, W, b):
    batch, seq, num_bins = transformer_logits.shape
    bs = batch * seq
    x2d = transformer_logits.reshape(bs, num_bins)
    b2d = b.reshape(1, _OUT_DIM)
    gumbel = _gumbel_noise()

    nsteps = num_bins // _BK
    bins_logits, resid, sel, selres = pl.pallas_call(
        functools.partial(_fused_body, nsteps=nsteps),
        grid=(nsteps, 2),
        in_specs=[
            pl.BlockSpec((bs, num_bins), lambda k, j: (0, 0)),
            pl.BlockSpec((_BK, _HALFW), lambda k, j: (k, j)),
            pl.BlockSpec((1, _OUT_DIM), lambda k, j: (0, 0)),
            pl.BlockSpec((bs, _BINS), lambda k, j: (0, 0)),
        ],
        out_specs=(
            pl.BlockSpec((bs, _BINS), lambda k, j: (0, 0)),
            pl.BlockSpec((bs, _OUT_DIM - _BINS), lambda k, j: (0, 0)),
            pl.BlockSpec((bs, 1), lambda k, j: (0, 0)),
            pl.BlockSpec((bs, _ADIM), lambda k, j: (0, 0)),
        ),
        out_shape=(
            jax.ShapeDtypeStruct((bs, _BINS), jnp.float32),
            jax.ShapeDtypeStruct((bs, _OUT_DIM - _BINS), jnp.float32),
            jax.ShapeDtypeStruct((bs, 1), jnp.int32),
            jax.ShapeDtypeStruct((bs, _ADIM), jnp.float32),
        ),
        compiler_params=pltpu.CompilerParams(
            dimension_semantics=("arbitrary", "arbitrary")
        ),
    )(x2d, W, b2d, gumbel)

    return (
        sel.reshape(batch, seq, 1),
        selres.reshape(batch, seq, _ADIM),
        resid.reshape(batch, seq, num_bins, _ADIM),
        bins_logits.reshape(batch, seq, num_bins),
    )
